# Initial kernel scaffold; baseline (speedup 1.0000x reference)
import jax
import jax.numpy as jnp
from jax.experimental import pallas as pl


def kernel(inp, head_w, head_b, t1_pw, t1_pb, t1_w, t1_b, t2_pw, t2_pb, t2_w, t2_b):
    raise NotImplementedError("write your pallas kernel here")


# speedup vs baseline: 1.8067x; 1.8067x over previous
import jax
import jax.numpy as jnp
from jax.experimental import pallas as pl

S = 2048
H = 768
HD = 4002
HOUT = 4000
D1, V1 = 192, 16000
D2, V2 = 48, 80000
V = HOUT + V1 + V2

RB = 128
CS = 3200

LW = 250
KB = 8
BW = KB * LW
R2 = 1024
NH = HOUT // BW
N1 = V1 // BW
N2 = V2 // BW

NEG = -1e30


def _stats_kernel(x_ref, hw_ref, hb_ref, p1w_ref, p1b_ref, t1w_ref, t1b_ref,
                  p2w_ref, p2b_ref, t2w_ref, t2b_ref,
                  uhead_ref, proj1_ref, proj2_ref, stats_ref):
    x = x_ref[:]
    zh = jnp.dot(x, hw_ref[:], preferred_element_type=jnp.float32)
    zh = zh + hb_ref[0, :][None, :]
    mh = jnp.max(zh, axis=-1, keepdims=True)
    eh = jnp.exp(zh - mh)
    sh = jnp.sum(eh, axis=-1, keepdims=True)
    inv_sh = 1.0 / sh
    for c in range(HOUT // LW):
        uhead_ref[:, c, :] = eh[:, c * LW:(c + 1) * LW]
    g1 = eh[:, HOUT:HOUT + 1] * inv_sh
    g2 = eh[:, HOUT + 1:HOUT + 2] * inv_sh

    p1 = jnp.dot(x, p1w_ref[:], preferred_element_type=jnp.float32)
    p1 = (p1 + p1b_ref[0, :][None, :]).astype(jnp.bfloat16)
    proj1_ref[:] = p1
    p2 = jnp.dot(x, p2w_ref[:], preferred_element_type=jnp.float32)
    p2 = (p2 + p2b_ref[0, :][None, :]).astype(jnp.bfloat16)
    proj2_ref[:] = p2

    def tail_stats(p, w_ref, b_ref, v):
        def body(i, carry):
            m, s = carry
            sl = pl.ds(i * CS, CS)
            z = jnp.dot(p, w_ref[:, sl], preferred_element_type=jnp.float32)
            z = z + b_ref[0, sl][None, :]
            mc = jnp.max(z, axis=-1, keepdims=True)
            mn = jnp.maximum(m, mc)
            s = s * jnp.exp(m - mn) + jnp.sum(jnp.exp(z - mn), axis=-1,
                                              keepdims=True)
            return mn, s
        m0 = jnp.full((p.shape[0], 1), NEG, dtype=jnp.float32)
        s0 = jnp.zeros((p.shape[0], 1), dtype=jnp.float32)
        return jax.lax.fori_loop(0, v // CS, body, (m0, s0))

    m1, s1 = tail_stats(p1, t1w_ref, t1b_ref, V1)
    m2, s2 = tail_stats(p2, t2w_ref, t2b_ref, V2)

    zeros = jnp.zeros_like(m1)
    stats_ref[:] = jnp.concatenate(
        [m1, g1 / s1, m2, g2 / s2, inv_sh, zeros, zeros, zeros], axis=1)


def _write_kernel(uhead_ref, proj1_ref, proj2_ref, stats_ref,
                  t1w_ref, t1b_ref, t2w_ref, t2b_ref, out_ref):
    j = pl.program_id(1)

    @pl.when(j < NH)
    def _head():
        inv_sh = stats_ref[:, 4:5]
        out_ref[:] = uhead_ref[:] * inv_sh[:, :, None]

    def tail(p, w_ref, b_ref, m, sc):
        for c in range(KB):
            z = jnp.dot(p, w_ref[:, c, :], preferred_element_type=jnp.float32)
            z = z + b_ref[0, c, :][None, :]
            out_ref[:, c, :] = jnp.exp(z - m) * sc

    @pl.when(jnp.logical_and(j >= NH, j < NH + N1))
    def _tail1():
        tail(proj1_ref[:], t1w_ref, t1b_ref,
             stats_ref[:, 0:1], stats_ref[:, 1:2])

    @pl.when(j >= NH + N1)
    def _tail2():
        tail(proj2_ref[:], t2w_ref, t2b_ref,
             stats_ref[:, 2:3], stats_ref[:, 3:4])


def kernel(inp, head_w, head_b, t1_pw, t1_pb, t1_w, t1_b,
           t2_pw, t2_pb, t2_w, t2_b):
    x = inp.reshape(S, H).astype(jnp.bfloat16)
    hw = head_w.astype(jnp.bfloat16)
    p1w = t1_pw.astype(jnp.bfloat16)
    t1wb = t1_w.astype(jnp.bfloat16)
    p2w = t2_pw.astype(jnp.bfloat16)
    t2wb = t2_w.astype(jnp.bfloat16)
    hb = head_b.reshape(1, HD)
    p1b = t1_pb.reshape(1, D1)
    t1b2 = t1_b.reshape(1, V1)
    p2b = t2_pb.reshape(1, D2)
    t2b2 = t2_b.reshape(1, V2)

    full = lambda shape: pl.BlockSpec(shape, lambda i: (0,) * len(shape))
    uhead, proj1, proj2, stats = pl.pallas_call(
        _stats_kernel,
        grid=(S // RB,),
        in_specs=[
            pl.BlockSpec((RB, H), lambda i: (i, 0)),
            full((H, HD)), full((1, HD)),
            full((H, D1)), full((1, D1)), full((D1, V1)), full((1, V1)),
            full((H, D2)), full((1, D2)), full((D2, V2)), full((1, V2)),
        ],
        out_specs=[
            pl.BlockSpec((RB, HOUT // LW, LW), lambda i: (i, 0, 0)),
            pl.BlockSpec((RB, D1), lambda i: (i, 0)),
            pl.BlockSpec((RB, D2), lambda i: (i, 0)),
            pl.BlockSpec((RB, 8), lambda i: (i, 0)),
        ],
        out_shape=[
            jax.ShapeDtypeStruct((S, HOUT // LW, LW), jnp.float32),
            jax.ShapeDtypeStruct((S, D1), jnp.bfloat16),
            jax.ShapeDtypeStruct((S, D2), jnp.bfloat16),
            jax.ShapeDtypeStruct((S, 8), jnp.float32),
        ],
    )(x, hw, hb, p1w, p1b, t1wb, t1b2, p2w, p2b, t2wb, t2b2)

    t1w3 = t1wb.reshape(D1, V1 // LW, LW)
    t2w3 = t2wb.reshape(D2, V2 // LW, LW)
    t1b3 = t1b2.reshape(1, V1 // LW, LW)
    t2b3 = t2b2.reshape(1, V2 // LW, LW)

    out = pl.pallas_call(
        _write_kernel,
        grid=(S // R2, NH + N1 + N2),
        in_specs=[
            pl.BlockSpec((R2, KB, LW),
                         lambda i, j: (i, jnp.minimum(j, NH - 1), 0)),
            pl.BlockSpec((R2, D1), lambda i, j: (i, 0)),
            pl.BlockSpec((R2, D2), lambda i, j: (i, 0)),
            pl.BlockSpec((R2, 8), lambda i, j: (i, 0)),
            pl.BlockSpec((D1, KB, LW),
                         lambda i, j: (0, jnp.clip(j - NH, 0, N1 - 1), 0)),
            pl.BlockSpec((1, KB, LW),
                         lambda i, j: (0, jnp.clip(j - NH, 0, N1 - 1), 0)),
            pl.BlockSpec((D2, KB, LW),
                         lambda i, j: (0, jnp.clip(j - NH - N1, 0, N2 - 1), 0)),
            pl.BlockSpec((1, KB, LW),
                         lambda i, j: (0, jnp.clip(j - NH - N1, 0, N2 - 1), 0)),
        ],
        out_specs=pl.BlockSpec((R2, KB, LW), lambda i, j: (i, j, 0)),
        out_shape=jax.ShapeDtypeStruct((S, V // LW, LW), jnp.float32),
    )(uhead, proj1, proj2, stats, t1w3, t1b3, t2w3, t2b3)

    return out.reshape(1, S, V)


# speedup vs baseline: 3.0842x; 1.7071x over previous
import jax
import jax.numpy as jnp
from jax.experimental import pallas as pl

S = 2048
H = 768
HD = 4002
HOUT = 4000
HPAD = 4096
D1, V1 = 192, 16000
D2, V2 = 48, 80000
V = HOUT + V1 + V2

RB = 128
CS = 3200

BW = 2048
NB = (V + BW - 1) // BW
R2 = 1024
A1_LO = 1 * BW
A1_W = 9 * BW
N1B = 9
A2_LO = 9 * BW
A2_W = 40 * BW
N2B = 40

NEG = -1e30


def _stats_kernel(x_ref, hw_ref, hb_ref, p1w_ref, p1b_ref, t1w_ref, t1b_ref,
                  p2w_ref, p2b_ref, t2w_ref, t2b_ref,
                  uhead_ref, proj1_ref, proj2_ref, stats_ref):
    x = x_ref[:]
    zh = jnp.dot(x, hw_ref[:], preferred_element_type=jnp.float32)
    zh = zh + hb_ref[0, :][None, :]
    mh = jnp.max(zh, axis=-1, keepdims=True)
    eh = jnp.exp(zh - mh)
    sh = jnp.sum(eh, axis=-1, keepdims=True)
    inv_sh = 1.0 / sh
    uhead_ref[:] = eh
    g1 = eh[:, HOUT:HOUT + 1] * inv_sh
    g2 = eh[:, HOUT + 1:HOUT + 2] * inv_sh

    p1 = jnp.dot(x, p1w_ref[:], preferred_element_type=jnp.float32)
    p1 = (p1 + p1b_ref[0, :][None, :]).astype(jnp.bfloat16)
    proj1_ref[:] = p1
    p2 = jnp.dot(x, p2w_ref[:], preferred_element_type=jnp.float32)
    p2 = (p2 + p2b_ref[0, :][None, :]).astype(jnp.bfloat16)
    proj2_ref[:] = p2

    def tail_stats(p, w_ref, b_ref, v):
        def body(i, carry):
            m, s = carry
            sl = pl.ds(i * CS, CS)
            z = jnp.dot(p, w_ref[:, sl], preferred_element_type=jnp.float32)
            z = z + b_ref[0, sl][None, :]
            mc = jnp.max(z, axis=-1, keepdims=True)
            mn = jnp.maximum(m, mc)
            s = s * jnp.exp(m - mn) + jnp.sum(jnp.exp(z - mn), axis=-1,
                                              keepdims=True)
            return mn, s
        m0 = jnp.full((p.shape[0], 1), NEG, dtype=jnp.float32)
        s0 = jnp.zeros((p.shape[0], 1), dtype=jnp.float32)
        return jax.lax.fori_loop(0, v // CS, body, (m0, s0))

    m1, s1 = tail_stats(p1, t1w_ref, t1b_ref, V1)
    m2, s2 = tail_stats(p2, t2w_ref, t2b_ref, V2)

    zeros = jnp.zeros_like(m1)
    stats_ref[:] = jnp.concatenate(
        [m1, g1 / s1, m2, g2 / s2, inv_sh, zeros, zeros, zeros], axis=1)


def _write_kernel(uh_ref, proj1_ref, proj2_ref, stats_ref,
                  a1w_ref, a1b_ref, a2w_ref, a2b_ref, out_ref):
    j = pl.program_id(1)

    def head_val():
        return uh_ref[:] * stats_ref[:, 4:5]

    def t1_val():
        z = jnp.dot(proj1_ref[:], a1w_ref[:],
                    preferred_element_type=jnp.float32)
        z = z + a1b_ref[0, :][None, :]
        return jnp.exp(z - stats_ref[:, 0:1]) * stats_ref[:, 1:2]

    def t2_val():
        z = jnp.dot(proj2_ref[:], a2w_ref[:],
                    preferred_element_type=jnp.float32)
        z = z + a2b_ref[0, :][None, :]
        return jnp.exp(z - stats_ref[:, 2:3]) * stats_ref[:, 3:4]

    def cols():
        return (j * BW
                + jax.lax.broadcasted_iota(jnp.int32, (1, BW), 1))

    @pl.when(j == 0)
    def _():
        out_ref[:] = head_val()

    @pl.when(j == 1)
    def _():
        out_ref[:] = jnp.where(cols() < HOUT, head_val(), t1_val())

    @pl.when(jnp.logical_and(j >= 2, j <= 8))
    def _():
        out_ref[:] = t1_val()

    @pl.when(j == 9)
    def _():
        out_ref[:] = jnp.where(cols() < HOUT + V1, t1_val(), t2_val())

    @pl.when(j >= 10)
    def _():
        out_ref[:] = t2_val()


def kernel(inp, head_w, head_b, t1_pw, t1_pb, t1_w, t1_b,
           t2_pw, t2_pb, t2_w, t2_b):
    x = inp.reshape(S, H).astype(jnp.bfloat16)
    hw = jnp.pad(head_w.astype(jnp.bfloat16), ((0, 0), (0, HPAD - HD)))
    hb = jnp.pad(head_b.reshape(1, HD), ((0, 0), (0, HPAD - HD)),
                 constant_values=NEG)
    p1w = t1_pw.astype(jnp.bfloat16)
    p2w = t2_pw.astype(jnp.bfloat16)
    p1b = t1_pb.reshape(1, D1)
    p2b = t2_pb.reshape(1, D2)
    t1wb = t1_w.astype(jnp.bfloat16)
    t2wb = t2_w.astype(jnp.bfloat16)
    t1b2 = t1_b.reshape(1, V1)
    t2b2 = t2_b.reshape(1, V2)

    full = lambda shape: pl.BlockSpec(shape, lambda i: (0,) * len(shape))
    uhead, proj1, proj2, stats = pl.pallas_call(
        _stats_kernel,
        grid=(S // RB,),
        in_specs=[
            pl.BlockSpec((RB, H), lambda i: (i, 0)),
            full((H, HPAD)), full((1, HPAD)),
            full((H, D1)), full((1, D1)), full((D1, V1)), full((1, V1)),
            full((H, D2)), full((1, D2)), full((D2, V2)), full((1, V2)),
        ],
        out_specs=[
            pl.BlockSpec((RB, HPAD), lambda i: (i, 0)),
            pl.BlockSpec((RB, D1), lambda i: (i, 0)),
            pl.BlockSpec((RB, D2), lambda i: (i, 0)),
            pl.BlockSpec((RB, 8), lambda i: (i, 0)),
        ],
        out_shape=[
            jax.ShapeDtypeStruct((S, HPAD), jnp.float32),
            jax.ShapeDtypeStruct((S, D1), jnp.bfloat16),
            jax.ShapeDtypeStruct((S, D2), jnp.bfloat16),
            jax.ShapeDtypeStruct((S, 8), jnp.float32),
        ],
    )(x, hw, hb, p1w, p1b, t1wb, t1b2, p2w, p2b, t2wb, t2b2)

    lo1 = HOUT - A1_LO
    a1w = jnp.pad(t1wb, ((0, 0), (lo1, A1_W - lo1 - V1)))
    a1b = jnp.pad(t1b2, ((0, 0), (lo1, A1_W - lo1 - V1)))
    lo2 = HOUT + V1 - A2_LO
    a2w = jnp.pad(t2wb, ((0, 0), (lo2, A2_W - lo2 - V2)))
    a2b = jnp.pad(t2b2, ((0, 0), (lo2, A2_W - lo2 - V2)))

    out = pl.pallas_call(
        _write_kernel,
        grid=(S // R2, NB),
        in_specs=[
            pl.BlockSpec((R2, BW), lambda i, j: (i, jnp.minimum(j, 1))),
            pl.BlockSpec((R2, D1), lambda i, j: (i, 0)),
            pl.BlockSpec((R2, D2), lambda i, j: (i, 0)),
            pl.BlockSpec((R2, 8), lambda i, j: (i, 0)),
            pl.BlockSpec((D1, BW), lambda i, j: (0, jnp.clip(j - 1, 0, N1B - 1))),
            pl.BlockSpec((1, BW), lambda i, j: (0, jnp.clip(j - 1, 0, N1B - 1))),
            pl.BlockSpec((D2, BW), lambda i, j: (0, jnp.clip(j - 9, 0, N2B - 1))),
            pl.BlockSpec((1, BW), lambda i, j: (0, jnp.clip(j - 9, 0, N2B - 1))),
        ],
        out_specs=pl.BlockSpec((R2, BW), lambda i, j: (i, j)),
        out_shape=jax.ShapeDtypeStruct((S, V), jnp.float32),
    )(uhead, proj1, proj2, stats, a1w, a1b, a2w, a2b)

    return out.reshape(1, S, V)
